# baseline (speedup 1.0000x reference)
import jax
import jax.numpy as jnp
from jax.experimental import pallas as pl
from jax.experimental.pallas import tpu as pltpu

_EPS = 1e-8


def _attn_kernel(scale_ref, c_ref, k_ref, v_ref, vhat_ref, alpha_ref, knorm_ref):
    qb = pl.program_id(1)

    @pl.when(qb == 0)
    def _normalize_keys():
        k = k_ref[...]
        k_n2 = jnp.sum(k * k, axis=-1, keepdims=True)
        knorm_ref[...] = k * jax.lax.rsqrt(jnp.maximum(k_n2, _EPS * _EPS))

    c = c_ref[...]
    c_n2 = jnp.sum(c * c, axis=-1, keepdims=True)
    c_norm = c * (scale_ref[0, 0] * jax.lax.rsqrt(jnp.maximum(c_n2, _EPS * _EPS)))

    s = jax.lax.dot_general(
        c_norm, knorm_ref[...],
        dimension_numbers=(((1,), (1,)), ((), ())),
        preferred_element_type=jnp.float32,
    )
    e = jnp.exp2(s)
    inv_z = 1.0 / jnp.sum(e, axis=-1, keepdims=True)

    r = jax.lax.dot_general(
        e, v_ref[...],
        dimension_numbers=(((1,), (0,)), ((), ())),
        preferred_element_type=jnp.float32,
    )

    alpha_ref[...] = e * inv_z
    vhat_ref[...] = r * inv_z


@jax.jit
def kernel(C, keys, vals, temp):
    B, N, D = C.shape
    W = keys.shape[1]
    NB = 128
    scale = (1.4426950408889634 / (temp + _EPS)).reshape(1, 1).astype(jnp.float32)

    vhat, alpha = pl.pallas_call(
        _attn_kernel,
        grid=(B, N // NB),
        in_specs=[
            pl.BlockSpec(memory_space=pltpu.SMEM),
            pl.BlockSpec((NB, D), lambda b, q: (b * (N // NB) + q, 0)),
            pl.BlockSpec((W, D), lambda b, q: (b, 0)),
            pl.BlockSpec((W, D), lambda b, q: (b, 0)),
        ],
        out_specs=[
            pl.BlockSpec((NB, D), lambda b, q: (b * (N // NB) + q, 0)),
            pl.BlockSpec((NB, W), lambda b, q: (b * (N // NB) + q, 0)),
        ],
        out_shape=[
            jax.ShapeDtypeStruct((B * N, D), jnp.float32),
            jax.ShapeDtypeStruct((B * N, W), jnp.float32),
        ],
        scratch_shapes=[pltpu.VMEM((W, D), jnp.float32)],
        compiler_params=pltpu.CompilerParams(
            dimension_semantics=("parallel", "arbitrary"),
        ),
    )(scale, C.reshape(B * N, D), keys.reshape(B * W, D), vals.reshape(B * W, D))
    return (vhat.reshape(B, N, D), alpha.reshape(B, N, W))
